# Initial kernel scaffold; baseline (speedup 1.0000x reference)
import functools

import jax
import jax.numpy as jnp
from jax import lax
from jax.experimental import pallas as pl
from jax.experimental.pallas import tpu as pltpu
from jax.experimental.pallas import tpu_sc as plsc

_NUM_CORES = 2
_NUM_SUBCORES = 16
_NUM_WORKERS = _NUM_CORES * _NUM_SUBCORES
_LANES = 16


@jax.jit
def _lr_pooled_lookup(xt, table_flat, bias16):
    F, B = xt.shape
    bpw = B // _NUM_WORKERS
    mesh = plsc.VectorSubcoreMesh(core_axis_name="c", subcore_axis_name="s")

    @functools.partial(
        pl.kernel,
        out_type=jax.ShapeDtypeStruct((B,), jnp.float32),
        mesh=mesh,
        scratch_types=[
            pltpu.VMEM((F, bpw), jnp.int32),
            pltpu.VMEM((F, bpw), jnp.float32),
            pltpu.VMEM((_LANES,), jnp.float32),
            pltpu.VMEM((bpw,), jnp.float32),
            pltpu.SemaphoreType.DMA,
        ],
    )
    def k(xt_hbm, tab_hbm, bias_hbm, out_hbm, xt_v, vals_v, bias_v, acc_v, gsem):
        wid = lax.axis_index("s") * _NUM_CORES + lax.axis_index("c")
        base = wid * bpw
        pltpu.sync_copy(xt_hbm.at[:, pl.ds(base, bpw)], xt_v)
        pltpu.sync_copy(bias_hbm, bias_v)
        copies = [
            pltpu.async_copy(tab_hbm.at[xt_v.at[f]], vals_v.at[f], gsem)
            for f in range(F)
        ]
        for c in copies:
            c.wait()
        bvec = bias_v[...]
        for i in range(bpw // _LANES):
            acc = bvec
            for f in range(F):
                acc = acc + vals_v[f, pl.ds(i * _LANES, _LANES)]
            acc_v[pl.ds(i * _LANES, _LANES)] = acc
        pltpu.sync_copy(acc_v, out_hbm.at[pl.ds(base, bpw)])

    return k(xt, table_flat, bias16)


def kernel(X, table, bias):
    B, F = X.shape
    xt = X.T
    out = _lr_pooled_lookup(xt, table.reshape(-1), jnp.broadcast_to(bias, (_LANES,)))
    return out.reshape(B, 1)


# speedup vs baseline: 1.4234x; 1.4234x over previous
import functools

import jax
import jax.numpy as jnp
from jax import lax
from jax.experimental import pallas as pl
from jax.experimental.pallas import tpu as pltpu
from jax.experimental.pallas import tpu_sc as plsc

_NUM_CORES = 2
_NUM_SUBCORES = 16
_NUM_WORKERS = _NUM_CORES * _NUM_SUBCORES
_LANES = 16
_CHUNK = 128


@jax.jit
def _lr_pooled_lookup(xt, table_flat, bias16):
    F, NW, NJ, C = xt.shape
    bpw = NJ * C
    B = NW * bpw
    mesh = plsc.VectorSubcoreMesh(core_axis_name="c", subcore_axis_name="s")

    @functools.partial(
        pl.kernel,
        out_type=jax.ShapeDtypeStruct((B,), jnp.float32),
        mesh=mesh,
        scratch_types=[
            pltpu.VMEM((F, NJ, C), jnp.int32),
            pltpu.VMEM((F, NJ, C), jnp.float32),
            pltpu.VMEM((_LANES,), jnp.float32),
            pltpu.VMEM((bpw,), jnp.float32),
            pltpu.SemaphoreType.DMA,
        ],
    )
    def k(xt_hbm, tab_hbm, bias_hbm, out_hbm, xt_v, vals_v, bias_v, acc_v, gsem):
        wid = lax.axis_index("s") * _NUM_CORES + lax.axis_index("c")
        base = wid * bpw
        pltpu.sync_copy(xt_hbm.at[:, wid], xt_v)
        pltpu.sync_copy(bias_hbm, bias_v)
        copies = [
            pltpu.async_copy(tab_hbm.at[xt_v.at[f, j]], vals_v.at[f, j], gsem)
            for f in range(F)
            for j in range(NJ)
        ]
        for c in copies:
            c.wait()
        bvec = bias_v[...]
        per_chunk = C // _LANES
        for i in range(bpw // _LANES):
            j, off = i // per_chunk, (i % per_chunk) * _LANES
            acc = bvec
            for f in range(F):
                acc = acc + vals_v[f, j, pl.ds(off, _LANES)]
            acc_v[pl.ds(i * _LANES, _LANES)] = acc
        pltpu.sync_copy(acc_v, out_hbm.at[pl.ds(base, bpw)])

    return k(xt, table_flat, bias16)


def kernel(X, table, bias):
    B, F = X.shape
    bpw = B // _NUM_WORKERS
    xt = X.T.reshape(F, _NUM_WORKERS, bpw // _CHUNK, _CHUNK)
    out = _lr_pooled_lookup(xt, table.reshape(-1), jnp.broadcast_to(bias, (_LANES,)))
    return out.reshape(B, 1)
